# Initial kernel scaffold; baseline (speedup 1.0000x reference)
import functools

import jax
import jax.numpy as jnp
from jax import lax
from jax.experimental import pallas as pl
from jax.experimental.pallas import tpu as pltpu
from jax.experimental.pallas import tpu_sc as plsc

LUT_DIM = 64
_NC = 2
_NS = 16
_NW = _NC * _NS
_CHUNK = 512


@functools.lru_cache(maxsize=None)
def _make_gather(b_total, dim):
    assert b_total % _NW == 0
    n_per_w = b_total // _NW
    chunk = _CHUNK
    assert n_per_w % chunk == 0
    n_chunks = n_per_w // chunk

    mesh = plsc.VectorSubcoreMesh(core_axis_name="c", subcore_axis_name="s")

    @functools.partial(
        pl.kernel,
        mesh=mesh,
        out_type=jax.ShapeDtypeStruct((b_total, dim), jnp.float32),
        scratch_types=[
            pltpu.VMEM((chunk,), jnp.int32),
            pltpu.VMEM((chunk, dim), jnp.float32),
            pltpu.SemaphoreType.DMA,
        ],
    )
    def gather_kernel(idx_hbm, table_hbm, out_hbm, idx_v, rows_v, sem):
        wid = lax.axis_index("s") * _NC + lax.axis_index("c")
        base = wid * n_per_w

        def body(i, carry):
            off = base + i * chunk
            pltpu.sync_copy(idx_hbm.at[pl.ds(off, chunk)], idx_v)
            pltpu.async_copy(table_hbm.at[idx_v], rows_v, sem).wait()
            pltpu.sync_copy(rows_v, out_hbm.at[pl.ds(off, chunk)])
            return carry

        lax.fori_loop(0, n_chunks, body, 0)

    return gather_kernel


def kernel(index, table):
    b, f = index.shape
    dim = table.shape[1]
    flat_idx = index.reshape(b * f).astype(jnp.int32)
    out = _make_gather(b * f, dim)(flat_idx, table)
    return out.reshape(b, f, dim)


# speedup vs baseline: 5.6584x; 5.6584x over previous
import functools

import jax
import jax.numpy as jnp
from jax import lax
from jax.experimental import pallas as pl
from jax.experimental.pallas import tpu as pltpu
from jax.experimental.pallas import tpu_sc as plsc

LUT_DIM = 64
_NC = 2
_NS = 16
_NW = _NC * _NS
_CHUNK = 512


@functools.lru_cache(maxsize=None)
def _make_gather(b_total, dim):
    assert b_total % _NW == 0
    n_per_w = b_total // _NW
    chunk = _CHUNK
    assert n_per_w % chunk == 0
    n_chunks = n_per_w // chunk
    assert n_chunks >= 2

    mesh = plsc.VectorSubcoreMesh(core_axis_name="c", subcore_axis_name="s")

    @functools.partial(
        pl.kernel,
        mesh=mesh,
        out_type=jax.ShapeDtypeStruct((b_total, dim), jnp.float32),
        scratch_types=[
            pltpu.VMEM((n_chunks, chunk), jnp.int32),
            pltpu.VMEM((2, chunk, dim), jnp.float32),
            pltpu.SemaphoreType.DMA,
            pltpu.SemaphoreType.DMA,
        ],
        compiler_params=pltpu.CompilerParams(use_tc_tiling_on_sc=False),
    )
    def gather_kernel(idx_hbm, table_hbm, out_hbm, idx_v, rows_v, gsem, wsem):
        wid = lax.axis_index("s") * _NC + lax.axis_index("c")
        base = wid * n_per_w
        pltpu.sync_copy(idx_hbm.at[wid], idx_v)

        def start_gather(c, slot):
            pltpu.async_copy(table_hbm.at[idx_v.at[c]], rows_v.at[slot], gsem)

        def wait_gather(c, slot):
            pltpu.make_async_copy(
                table_hbm.at[idx_v.at[c]], rows_v.at[slot], gsem
            ).wait()

        def start_write(c, slot):
            pltpu.async_copy(
                rows_v.at[slot], out_hbm.at[pl.ds(base + c * chunk, chunk)], wsem
            )

        def wait_write(c, slot):
            pltpu.make_async_copy(
                rows_v.at[slot], out_hbm.at[pl.ds(base + c * chunk, chunk)], wsem
            ).wait()

        start_gather(0, 0)

        def body(i, carry):
            slot = lax.rem(i, 2)
            nslot = 1 - slot

            @pl.when(i + 1 < n_chunks)
            def _():
                @pl.when(i >= 1)
                def _():
                    wait_write(i - 1, nslot)

                start_gather(i + 1, nslot)

            wait_gather(i, slot)
            start_write(i, slot)
            return carry

        lax.fori_loop(0, n_chunks, body, 0)
        wait_write(n_chunks - 2, n_chunks % 2)
        wait_write(n_chunks - 1, (n_chunks - 1) % 2)

    return gather_kernel


def kernel(index, table):
    b, f = index.shape
    dim = table.shape[1]
    flat_idx = index.reshape(b * f).astype(jnp.int32)
    n_per_w = (b * f) // _NW
    idx3 = flat_idx.reshape(_NW, n_per_w // _CHUNK, _CHUNK)
    out = _make_gather(b * f, dim)(idx3, table)
    return out.reshape(b, f, dim)


# speedup vs baseline: 5.6622x; 1.0007x over previous
import functools

import jax
import jax.numpy as jnp
from jax import lax
from jax.experimental import pallas as pl
from jax.experimental.pallas import tpu as pltpu
from jax.experimental.pallas import tpu_sc as plsc

LUT_DIM = 64
_NC = 2
_NS = 16
_NW = _NC * _NS
_CHUNK = 256
_RING = 4


@functools.lru_cache(maxsize=None)
def _make_gather(b_total, dim):
    assert b_total % _NW == 0
    n_per_w = b_total // _NW
    chunk = _CHUNK
    ring = _RING
    assert n_per_w % chunk == 0
    n_chunks = n_per_w // chunk
    assert n_chunks >= ring

    mesh = plsc.VectorSubcoreMesh(core_axis_name="c", subcore_axis_name="s")

    @functools.partial(
        pl.kernel,
        mesh=mesh,
        out_type=jax.ShapeDtypeStruct((b_total, dim), jnp.float32),
        scratch_types=[
            pltpu.VMEM((n_chunks, chunk), jnp.int32),
            pltpu.VMEM((ring, chunk, dim), jnp.float32),
            pltpu.SemaphoreType.DMA,
            pltpu.SemaphoreType.DMA,
        ],
        compiler_params=pltpu.CompilerParams(use_tc_tiling_on_sc=False),
    )
    def gather_kernel(idx_hbm, table_hbm, out_hbm, idx_v, rows_v, gsem, wsem):
        wid = lax.axis_index("s") * _NC + lax.axis_index("c")
        base = wid * n_per_w
        pltpu.sync_copy(idx_hbm.at[wid], idx_v)

        def start_gather(c, slot):
            pltpu.async_copy(table_hbm.at[idx_v.at[c]], rows_v.at[slot], gsem)

        def wait_gather(c, slot):
            pltpu.make_async_copy(
                table_hbm.at[idx_v.at[c]], rows_v.at[slot], gsem
            ).wait()

        def start_write(c, slot):
            pltpu.async_copy(
                rows_v.at[slot], out_hbm.at[pl.ds(base + c * chunk, chunk)], wsem
            )

        def wait_write(c, slot):
            pltpu.make_async_copy(
                rows_v.at[slot], out_hbm.at[pl.ds(base + c * chunk, chunk)], wsem
            ).wait()

        for c in range(ring - 1):
            start_gather(c, c)

        def body(i, carry):
            slot = lax.rem(i, ring)
            g = i + ring - 1
            gslot = lax.rem(g, ring)

            @pl.when(g < n_chunks)
            def _():
                @pl.when(i >= 1)
                def _():
                    wait_write(i - 1, lax.rem(i - 1, ring))

                start_gather(g, gslot)

            wait_gather(i, slot)
            start_write(i, slot)
            return carry

        lax.fori_loop(0, n_chunks, body, 0)
        for c in range(n_chunks - ring, n_chunks):
            wait_write(c, c % ring)

    return gather_kernel


def kernel(index, table):
    b, f = index.shape
    dim = table.shape[1]
    flat_idx = index.reshape(b * f).astype(jnp.int32)
    n_per_w = (b * f) // _NW
    idx3 = flat_idx.reshape(_NW, n_per_w // _CHUNK, _CHUNK)
    out = _make_gather(b * f, dim)(idx3, table)
    return out.reshape(b, f, dim)


# speedup vs baseline: 5.6692x; 1.0012x over previous
import functools

import jax
import jax.numpy as jnp
from jax import lax
from jax.experimental import pallas as pl
from jax.experimental.pallas import tpu as pltpu
from jax.experimental.pallas import tpu_sc as plsc

LUT_DIM = 64
_NC = 2
_NS = 16
_NW = _NC * _NS
_CHUNK = 512
_RING = 3


@functools.lru_cache(maxsize=None)
def _make_gather(b_total, dim):
    assert b_total % _NW == 0
    n_per_w = b_total // _NW
    chunk = _CHUNK
    ring = _RING
    assert n_per_w % chunk == 0
    n_chunks = n_per_w // chunk
    assert n_chunks >= ring

    mesh = plsc.VectorSubcoreMesh(core_axis_name="c", subcore_axis_name="s")

    @functools.partial(
        pl.kernel,
        mesh=mesh,
        out_type=jax.ShapeDtypeStruct((b_total, dim), jnp.float32),
        scratch_types=[
            pltpu.VMEM((ring, chunk), jnp.int32),
            pltpu.VMEM((ring, chunk, dim), jnp.float32),
            pltpu.SemaphoreType.DMA,
            pltpu.SemaphoreType.DMA,
            pltpu.SemaphoreType.DMA,
        ],
        compiler_params=pltpu.CompilerParams(use_tc_tiling_on_sc=False),
    )
    def gather_kernel(idx_hbm, table_hbm, out_hbm, idx_v, rows_v, isem, gsem, wsem):
        cid = lax.axis_index("c")
        sid = lax.axis_index("s")
        wid = sid * _NC + cid
        base = wid * n_per_w

        def start_idx(c):
            pltpu.async_copy(
                idx_hbm.at[wid, c], idx_v.at[lax.rem(c, ring)], isem
            )

        def wait_idx(c):
            pltpu.make_async_copy(
                idx_hbm.at[wid, c], idx_v.at[lax.rem(c, ring)], isem
            ).wait()

        def start_gather(c):
            slot = lax.rem(c, ring)
            pltpu.async_copy(table_hbm.at[idx_v.at[slot]], rows_v.at[slot], gsem)

        def wait_gather(c):
            slot = lax.rem(c, ring)
            pltpu.make_async_copy(
                table_hbm.at[idx_v.at[slot]], rows_v.at[slot], gsem
            ).wait()

        def start_write(c):
            pltpu.async_copy(
                rows_v.at[lax.rem(c, ring)],
                out_hbm.at[pl.ds(base + c * chunk, chunk)],
                wsem,
            )

        def wait_write(c):
            pltpu.make_async_copy(
                rows_v.at[lax.rem(c, ring)],
                out_hbm.at[pl.ds(base + c * chunk, chunk)],
                wsem,
            ).wait()

        start_idx(0)
        start_idx(1)
        wait_idx(0)
        start_gather(0)

        def body(i, carry):
            @pl.when(i + 2 < n_chunks)
            def _():
                start_idx(i + 2)

            @pl.when(i + 1 < n_chunks)
            def _():
                @pl.when(i + 1 >= ring)
                def _():
                    wait_write(i + 1 - ring)

                wait_idx(i + 1)
                start_gather(i + 1)

            wait_gather(i)
            start_write(i)
            return carry

        lax.fori_loop(0, n_chunks, body, 0)
        for c in range(n_chunks - ring, n_chunks):
            wait_write(c)

    return gather_kernel


def kernel(index, table):
    b, f = index.shape
    dim = table.shape[1]
    flat_idx = index.reshape(b * f).astype(jnp.int32)
    n_per_w = (b * f) // _NW
    idx3 = flat_idx.reshape(_NW, n_per_w // _CHUNK, _CHUNK)
    out = _make_gather(b * f, dim)(idx3, table)
    return out.reshape(b, f, dim)
